# baseline (speedup 1.0000x reference)
import functools

import jax
import jax.numpy as jnp
from jax import lax
from jax.experimental import pallas as pl
from jax.experimental.pallas import tpu as pltpu
from jax.experimental.pallas import tpu_sc as plsc

_info = plsc.get_sparse_core_info()
_NC, _NS, _L = _info.num_cores, _info.num_subcores, _info.num_lanes
_NW = _NC * _NS


def _make_gather(V, D, N, chunk):
    n_per_w = N // _NW
    n_chunks = n_per_w // chunk
    C4 = chunk // 4
    mesh = plsc.VectorSubcoreMesh(core_axis_name="c", subcore_axis_name="s")

    @functools.partial(
        pl.kernel,
        mesh=mesh,
        out_type=jax.ShapeDtypeStruct((N // 4, 4 * D), jnp.float32),
        compiler_params=pltpu.CompilerParams(use_tc_tiling_on_sc=False),
        scratch_types=[
            pltpu.VMEM((2, chunk), jnp.int32),
            pltpu.VMEM((2, 4, C4, D), jnp.float32),
            pltpu.SemaphoreType.DMA,
            pltpu.SemaphoreType.DMA,
            pltpu.SemaphoreType.DMA,
            pltpu.SemaphoreType.DMA,
        ],
    )
    def gather(ids_hbm, table_hbm, out_hbm, idx_v, rows_v, g0, g1, o0, o1):
        gsems = (g0, g1)
        osems = (o0, o1)
        wid = lax.axis_index("s") * _NC + lax.axis_index("c")
        base = wid * n_per_w
        base4 = base // 4

        def stage(i, b):
            off = base + i * chunk
            pltpu.sync_copy(ids_hbm.at[pl.ds(off, chunk)], idx_v.at[b])
            for q in range(4):
                pltpu.async_copy(
                    table_hbm.at[idx_v.at[b, pl.ds(q * C4, C4)]],
                    rows_v.at[b, q],
                    gsems[b],
                )

        stage(0, 0)

        def body(i2, carry):
            for b in range(2):
                i = i2 * 2 + b
                if b == 0:
                    stage(i + 1, 1)
                else:
                    @pl.when(i2 < n_chunks // 2 - 1)
                    def _():
                        stage(i + 1, 0)
                for q in range(4):
                    pltpu.make_async_copy(
                        table_hbm.at[idx_v.at[b, pl.ds(q * C4, C4)]],
                        rows_v.at[b, q],
                        gsems[b],
                    ).wait()
                off4 = base4 + i * C4

                @pl.when(i2 >= 1)
                def _():
                    for q in range(4):
                        pltpu.make_async_copy(
                            rows_v.at[b, q],
                            out_hbm.at[pl.ds(off4, C4), pl.ds(q * D, D)],
                            osems[b],
                        ).wait()

                for q in range(4):
                    pltpu.async_copy(
                        rows_v.at[b, q],
                        out_hbm.at[pl.ds(off4, C4), pl.ds(q * D, D)],
                        osems[b],
                    )
            return carry

        lax.fori_loop(0, n_chunks // 2, body, 0)
        for b in range(2):
            off4 = base4 + (n_chunks - 2 + b) * C4
            for q in range(4):
                pltpu.make_async_copy(
                    rows_v.at[b, q],
                    out_hbm.at[pl.ds(off4, C4), pl.ds(q * D, D)],
                    osems[b],
                ).wait()

    return gather


def kernel(ids_bt, emb_weight):
    B, T = ids_bt.shape
    V, D = emb_weight.shape
    N = B * T
    ids_flat = (
        ids_bt.reshape(_NW, N // (_NW * 1600), 400, 4)
        .transpose(0, 1, 3, 2)
        .reshape(N)
        .astype(jnp.int32)
    )
    table_q = lax.optimization_barrier(emb_weight.reshape(V * D // 128, 128))
    table = table_q.reshape(V, D)
    out = _make_gather(V, D, N, 1600)(ids_flat, table)
    return out.reshape(B, T, D)


# speedup vs baseline: 1.0242x; 1.0242x over previous
import functools

import jax
import jax.numpy as jnp
from jax import lax
from jax.experimental import pallas as pl
from jax.experimental.pallas import tpu as pltpu
from jax.experimental.pallas import tpu_sc as plsc

_info = plsc.get_sparse_core_info()
_NC, _NS = _info.num_cores, _info.num_subcores
_NW = _NC * _NS


def _make_gather(V, D, N, chunk):
    n_per_w = N // _NW
    n_chunks = n_per_w // chunk
    mesh = plsc.VectorSubcoreMesh(core_axis_name="c", subcore_axis_name="s")

    @functools.partial(
        pl.kernel,
        mesh=mesh,
        out_type=jax.ShapeDtypeStruct((N, D), jnp.float32),
        compiler_params=pltpu.CompilerParams(use_tc_tiling_on_sc=False),
        scratch_types=[
            pltpu.VMEM((2, chunk), jnp.int32),
            pltpu.VMEM((2, chunk, D), jnp.float32),
            pltpu.SemaphoreType.DMA,
            pltpu.SemaphoreType.DMA,
            pltpu.SemaphoreType.DMA,
            pltpu.SemaphoreType.DMA,
        ],
    )
    def gather(ids_hbm, table_hbm, out_hbm, idx_v, rows_v, g0, g1, o0, o1):
        gsems = (g0, g1)
        osems = (o0, o1)
        wid = lax.axis_index("s") * _NC + lax.axis_index("c")
        base = wid * n_per_w

        def stage(i, b):
            off = base + i * chunk
            pltpu.sync_copy(ids_hbm.at[pl.ds(off, chunk)], idx_v.at[b])
            pltpu.async_copy(table_hbm.at[idx_v.at[b]], rows_v.at[b], gsems[b])

        def wait_write(i, b):
            off = base + i * chunk
            pltpu.make_async_copy(
                rows_v.at[b], out_hbm.at[pl.ds(off, chunk)], osems[b]
            ).wait()

        stage(0, 0)

        def body(i2, carry):
            for b in range(2):
                i = i2 * 2 + b
                if b == 0:
                    @pl.when(i2 >= 1)
                    def _():
                        wait_write(i - 1, 1)
                    stage(i + 1, 1)
                else:
                    @pl.when(i2 < n_chunks // 2 - 1)
                    def _():
                        wait_write(i - 1, 0)
                        stage(i + 1, 0)
                pltpu.make_async_copy(
                    table_hbm.at[idx_v.at[b]], rows_v.at[b], gsems[b]
                ).wait()
                off = base + i * chunk
                pltpu.async_copy(
                    rows_v.at[b], out_hbm.at[pl.ds(off, chunk)], osems[b]
                )
            return carry

        lax.fori_loop(0, n_chunks // 2, body, 0)
        for b in range(2):
            off = base + (n_chunks - 2 + b) * chunk
            pltpu.make_async_copy(
                rows_v.at[b], out_hbm.at[pl.ds(off, chunk)], osems[b]
            ).wait()

    return gather


def kernel(ids_bt, emb_weight):
    B, T = ids_bt.shape
    V, D = emb_weight.shape
    N = B * T
    ids_flat = ids_bt.reshape(N).astype(jnp.int32)
    table_q = lax.optimization_barrier(emb_weight.reshape(V * D // 128, 128))
    table = table_q.reshape(V, D)
    out = _make_gather(V, D, N, 1600)(ids_flat, table)
    return out.reshape(B, T, D)
